# Initial kernel scaffold; baseline (speedup 1.0000x reference)
import jax
import jax.numpy as jnp
from jax.experimental import pallas as pl


def kernel(x, hidden_attr, W_hid, W_src, W_e1, W_e2, Wp1, Wp2, W_d1, W_d2, W_out, enc_src, enc_dst, proc_src, proc_dst, dec_src, dec_dst):
    raise NotImplementedError("write your pallas kernel here")


# speedup vs baseline: 1.0850x; 1.0850x over previous
import functools

import jax
import jax.numpy as jnp
from jax import lax
from jax.experimental import pallas as pl
from jax.experimental.pallas import tpu as pltpu
from jax.experimental.pallas import tpu_sc as plsc

C = 256
COLS = 128
_CHUNK = 128


def _mm(x, w, res=None, relu=False, bm=2000, m_out=None, interpret=False):
    M, K = x.shape
    if m_out is not None:
        M = m_out
    N = w.shape[1]
    grid = (pl.cdiv(M, bm),)

    def body(*refs):
        if res is not None:
            x_ref, w_ref, r_ref, o_ref = refs
        else:
            x_ref, w_ref, o_ref = refs
        acc = jnp.dot(x_ref[...], w_ref[...], preferred_element_type=jnp.float32)
        if res is not None:
            acc = acc + r_ref[...]
        if relu:
            acc = jnp.maximum(acc, 0.0)
        o_ref[...] = acc

    in_specs = [
        pl.BlockSpec((bm, K), lambda m: (m, 0)),
        pl.BlockSpec((K, N), lambda m: (0, 0)),
    ]
    args = [x, w]
    if res is not None:
        in_specs.append(pl.BlockSpec((bm, N), lambda m: (m, 0)))
        args.append(res)
    return pl.pallas_call(
        body,
        grid=grid,
        in_specs=in_specs,
        out_specs=pl.BlockSpec((bm, N), lambda m: (m, 0)),
        out_shape=jax.ShapeDtypeStruct((M, N), jnp.float32),
        interpret=interpret,
    )(*args)


def _sc_edge(tabP, tabQ, src, dst, n_dst, n_ranges):
    NQ = tabQ.shape[0]
    E = src.shape[0]
    mesh = plsc.VectorSubcoreMesh(core_axis_name="c", subcore_axis_name="s")
    NS = mesh.num_subcores

    rpt = (-(-n_dst // (n_ranges * NS)) + 7) // 8 * 8
    R = rpt * NS
    nfull, rem = rpt // _CHUNK, rpt % _CHUNK
    n_chunks = -(-E // (NS * _CHUNK))
    n_chunks = (n_chunks + 7) // 8 * 8
    ns8 = n_chunks // 8
    E_pad = NS * n_chunks * _CHUNK
    BR = _CHUNK if n_ranges == 1 else _CHUNK // 2

    pad = E_pad - E
    src_p = jnp.concatenate(
        [src.astype(jnp.int32), jnp.zeros((pad,), jnp.int32)])
    dst_p = jnp.concatenate(
        [dst.astype(jnp.int32), jnp.full((pad,), n_ranges * R, jnp.int32)])
    src3 = src_p.reshape(NS, n_chunks, _CHUNK)
    dst3 = dst_p.reshape(NS, n_chunks, _CHUNK)

    @functools.partial(
        pl.kernel,
        mesh=mesh,
        out_type=jax.ShapeDtypeStruct((n_ranges * R, 2 * COLS), jnp.float32),
        scratch_types=[
            pltpu.VMEM((8, _CHUNK), jnp.int32),
            pltpu.VMEM((8, _CHUNK), jnp.int32),
            pltpu.VMEM((_CHUNK,), jnp.int32),
            pltpu.VMEM((_CHUNK,), jnp.int32),
            pltpu.VMEM((_CHUNK, COLS), jnp.float32),
            pltpu.VMEM((BR, COLS), jnp.float32),
            pltpu.VMEM_SHARED((R + 8, COLS), jnp.float32),
            pltpu.SemaphoreType.DMA,
            pltpu.SemaphoreType.DMA,
        ],
    )
    def k(tabP_h, tabQ_h, src_h, dst_h, out_h,
          srcwin, dstwin, adjwin, gathwin, bufA, bufB, acc, semA, semB):
        c = lax.axis_index("c")
        t = lax.axis_index("s")
        cs = pl.ds(c * COLS, COLS)
        base = t * rpt

        def combine(a_off, b_off, n):
            @pl.loop(0, n, unroll=1)
            def rows(q):
                ra = a_off + q
                rb = b_off + q
                for kk in range(COLS // 16):
                    sl = pl.ds(kk * 16, 16)
                    bufA[ra, sl] = jnp.maximum(
                        bufA[ra, sl] + bufB[rb, sl], 0.0)

        for p in range(n_ranges):
            def zrow(r, carry):
                for kk in range(COLS // 16):
                    bufA[r, pl.ds(kk * 16, 16)] = jnp.zeros((16,), jnp.float32)
                return carry
            lax.fori_loop(0, _CHUNK, zrow, 0)

            def zero_blk(z, carry):
                pltpu.sync_copy(bufA,
                                acc.at[pl.ds(base + z * _CHUNK, _CHUNK)])
                return carry
            lax.fori_loop(0, nfull, zero_blk, 0)
            if rem:
                pltpu.sync_copy(bufA.at[pl.ds(0, rem)],
                                acc.at[pl.ds(base + nfull * _CHUNK, rem)])
            plsc.subcore_barrier()

            def superchunk(j8, carry):
                pltpu.sync_copy(src_h.at[t].at[pl.ds(j8 * 8, 8)], srcwin)
                pltpu.sync_copy(dst_h.at[t].at[pl.ds(j8 * 8, 8)], dstwin)

                def chunk(r, carry2):
                    cpA = pltpu.async_copy(
                        tabP_h.at[srcwin.at[r], cs], bufA, semA)
                    for v in range(_CHUNK // 16):
                        sl = pl.ds(v * 16, 16)
                        d = dstwin[r, sl]
                        a = d - p * R
                        ok = (a >= 0) & (a < R)
                        adjwin[sl] = jnp.where(ok, a, R)
                        gathwin[sl] = jnp.minimum(d, NQ - 1)
                    if BR == _CHUNK:
                        cpB = pltpu.async_copy(
                            tabQ_h.at[gathwin, cs], bufB, semB)
                        cpA.wait()
                        cpB.wait()
                        combine(0, 0, _CHUNK)
                    else:
                        cpB = pltpu.async_copy(
                            tabQ_h.at[gathwin.at[pl.ds(0, BR)], cs],
                            bufB, semB)
                        cpA.wait()
                        cpB.wait()
                        combine(0, 0, BR)
                        cpB2 = pltpu.async_copy(
                            tabQ_h.at[gathwin.at[pl.ds(BR, BR)], cs],
                            bufB, semB)
                        cpB2.wait()
                        combine(BR, 0, BR)
                    pltpu.sync_copy(bufA, acc.at[adjwin], add=True)
                    return carry2
                lax.fori_loop(0, 8, chunk, 0)
                return carry
            lax.fori_loop(0, ns8, superchunk, 0)
            plsc.subcore_barrier()

            def dump_blk(z, carry):
                off = base + z * _CHUNK
                pltpu.sync_copy(acc.at[pl.ds(off, _CHUNK)], bufA)
                pltpu.sync_copy(
                    bufA, out_h.at[pl.ds(p * R + off, _CHUNK), cs])
                return carry
            lax.fori_loop(0, nfull, dump_blk, 0)
            if rem:
                off = base + nfull * _CHUNK
                pltpu.sync_copy(acc.at[pl.ds(off, rem)],
                                bufA.at[pl.ds(0, rem)])
                pltpu.sync_copy(bufA.at[pl.ds(0, rem)],
                                out_h.at[pl.ds(p * R + off, rem), cs])
            plsc.subcore_barrier()

    return k(tabP, tabQ, src3, dst3)


def kernel(x, hidden_attr, W_hid, W_src, W_e1, W_e2, Wp1, Wp2, W_d1, W_d2,
           W_out, enc_src, enc_dst, proc_src, proc_dst, dec_src, dec_dst):
    n_hid = hidden_attr.shape[0]
    n_data = x.shape[0]
    n_layers = Wp1.shape[0]

    xe = _mm(x, W_src, relu=True)
    h = _mm(hidden_attr, W_hid, relu=True)

    P = _mm(xe, W_e1[:C])
    Q = _mm(h, W_e1[C:])
    S = _sc_edge(P, Q, enc_src, enc_dst, n_hid, 1)
    h = _mm(S, W_e2, res=h, relu=True, m_out=n_hid)

    for k in range(n_layers):
        A = _mm(h, Wp1[k][:C])
        B = _mm(h, Wp1[k][C:])
        S = _sc_edge(A, B, proc_src, proc_dst, n_hid, 1)
        h = _mm(S, Wp2[k], res=h, m_out=n_hid)

    Pd = _mm(h, W_d1[:C])
    Qd = _mm(xe, W_d1[C:])
    Sd = _sc_edge(Pd, Qd, dec_src, dec_dst, n_data, 4)
    xd = _mm(Sd, W_d2, res=xe, m_out=n_data)

    return _mm(xd, W_out)


# speedup vs baseline: 1.5932x; 1.4684x over previous
import functools

import jax
import jax.numpy as jnp
from jax import lax
from jax.experimental import pallas as pl
from jax.experimental.pallas import tpu as pltpu
from jax.experimental.pallas import tpu_sc as plsc

C = 256
COLS = 128
_CHUNK = 128


def _mm(x, w, res=None, relu=False, bm=2000, m_out=None, interpret=False):
    M, K = x.shape
    if m_out is not None:
        M = m_out
    N = w.shape[1]
    grid = (pl.cdiv(M, bm),)

    def body(*refs):
        if res is not None:
            x_ref, w_ref, r_ref, o_ref = refs
        else:
            x_ref, w_ref, o_ref = refs
        acc = jnp.dot(x_ref[...], w_ref[...], preferred_element_type=jnp.float32)
        if res is not None:
            acc = acc + r_ref[...]
        if relu:
            acc = jnp.maximum(acc, 0.0)
        o_ref[...] = acc

    in_specs = [
        pl.BlockSpec((bm, K), lambda m: (m, 0)),
        pl.BlockSpec((K, N), lambda m: (0, 0)),
    ]
    args = [x, w]
    if res is not None:
        in_specs.append(pl.BlockSpec((bm, N), lambda m: (m, 0)))
        args.append(res)
    return pl.pallas_call(
        body,
        grid=grid,
        in_specs=in_specs,
        out_specs=pl.BlockSpec((bm, N), lambda m: (m, 0)),
        out_shape=jax.ShapeDtypeStruct((M, N), jnp.float32),
        interpret=interpret,
    )(*args)


def _sc_edge(tabP, tabQ, src, dst, n_dst, n_ranges):
    NQ = tabQ.shape[0]
    E = src.shape[0]
    mesh = plsc.VectorSubcoreMesh(core_axis_name="c", subcore_axis_name="s")
    NS = mesh.num_subcores

    rpt = (-(-n_dst // (n_ranges * NS)) + 7) // 8 * 8
    R = rpt * NS
    n_chunks = -(-E // (NS * _CHUNK))
    n_chunks = (n_chunks + 7) // 8 * 8
    ns8 = n_chunks // 8
    E_pad = NS * n_chunks * _CHUNK
    GC = 64 if n_ranges == 1 else 32
    SUB = _CHUNK // GC
    NG = 8 * SUB
    nfull, rem = rpt // GC, rpt % GC

    pad = E_pad - E
    src_p = jnp.concatenate(
        [src.astype(jnp.int32), jnp.zeros((pad,), jnp.int32)])
    dst_p = jnp.concatenate(
        [dst.astype(jnp.int32), jnp.full((pad,), n_ranges * R, jnp.int32)])
    src3 = src_p.reshape(NS, n_chunks, _CHUNK)
    dst3 = dst_p.reshape(NS, n_chunks, _CHUNK)

    @functools.partial(
        pl.kernel,
        mesh=mesh,
        out_type=jax.ShapeDtypeStruct((n_ranges * R, 2 * COLS), jnp.float32),
        scratch_types=[
            pltpu.VMEM((8, _CHUNK), jnp.int32),
            pltpu.VMEM((8, _CHUNK), jnp.int32),
            pltpu.VMEM((2, GC), jnp.int32),
            pltpu.VMEM((2, GC), jnp.int32),
            pltpu.VMEM((2, GC), jnp.int32),
            pltpu.VMEM((2, GC, COLS), jnp.float32),
            pltpu.VMEM((2, GC, COLS), jnp.float32),
            pltpu.VMEM_SHARED((R + 8, COLS), jnp.float32),
            [pltpu.SemaphoreType.DMA] * 2,
            [pltpu.SemaphoreType.DMA] * 2,
        ],
    )
    def k(tabP_h, tabQ_h, src_h, dst_h, out_h,
          srcwin, dstwin, adjwin, gathwin, srcw2, bufA, bufB, acc,
          semA, semB):
        c = lax.axis_index("c")
        t = lax.axis_index("s")
        cs = pl.ds(c * COLS, COLS)
        base = t * rpt
        dummy = R + (t & 7)

        def prep(g, q):
            r = g // SUB
            h = g % SUB
            for v in range(GC // 16):
                slw = pl.ds(h * GC + v * 16, 16)
                slq = pl.ds(v * 16, 16)
                d = dstwin[r, slw]
                a = d - p * R
                ok = (a >= 0) & (a < R)
                adjwin[q, slq] = jnp.where(ok, a, dummy)
                gathwin[q, slq] = jnp.minimum(d, NQ - 1)
                srcw2[q, slq] = srcwin[r, slw]

        def issue(g, q):
            pltpu.async_copy(
                tabP_h.at[srcw2.at[q], cs], bufA.at[q], semA[q])
            pltpu.async_copy(
                tabQ_h.at[gathwin.at[q], cs], bufB.at[q], semB[q])

        def wait_gathers(q):
            pltpu.make_async_copy(
                tabP_h.at[srcw2.at[q], cs], bufA.at[q], semA[q]).wait()
            pltpu.make_async_copy(
                tabQ_h.at[gathwin.at[q], cs], bufB.at[q], semB[q]).wait()

        def combine(q):
            @pl.loop(0, GC, unroll=1)
            def rows(rr):
                for kk in range(COLS // 16):
                    sl = pl.ds(kk * 16, 16)
                    bufA[q, rr, sl] = jnp.maximum(
                        bufA[q, rr, sl] + bufB[q, rr, sl], 0.0)

        for p in range(n_ranges):
            def zrow(r, carry):
                for kk in range(COLS // 16):
                    bufA[0, r, pl.ds(kk * 16, 16)] = jnp.zeros((16,),
                                                               jnp.float32)
                return carry
            lax.fori_loop(0, GC, zrow, 0)

            def zero_blk(z, carry):
                pltpu.sync_copy(bufA.at[0], acc.at[pl.ds(base + z * GC, GC)])
                return carry
            lax.fori_loop(0, nfull, zero_blk, 0)
            if rem:
                pltpu.sync_copy(bufA.at[0].at[pl.ds(0, rem)],
                                acc.at[pl.ds(base + nfull * GC, rem)])
            plsc.subcore_barrier()

            def superchunk(j8, carry):
                pltpu.sync_copy(src_h.at[t].at[pl.ds(j8 * 8, 8)], srcwin)
                pltpu.sync_copy(dst_h.at[t].at[pl.ds(j8 * 8, 8)], dstwin)
                prep(0, 0)
                issue(0, 0)

                def pair(u, carry2):
                    for q in (0, 1):
                        g = 2 * u + q

                        @pl.when(g + 1 < NG)
                        def _pf():
                            prep(g + 1, 1 - q)
                            issue(g + 1, 1 - q)
                        wait_gathers(q)
                        combine(q)
                        pltpu.sync_copy(bufA.at[q], acc.at[adjwin.at[q]],
                                        add=True)
                    return carry2
                lax.fori_loop(0, NG // 2, pair, 0)
                return carry
            lax.fori_loop(0, ns8, superchunk, 0)
            plsc.subcore_barrier()

            def dump_blk(z, carry):
                off = base + z * GC
                pltpu.sync_copy(acc.at[pl.ds(off, GC)], bufA.at[0])
                pltpu.sync_copy(
                    bufA.at[0], out_h.at[pl.ds(p * R + off, GC), cs])
                return carry
            lax.fori_loop(0, nfull, dump_blk, 0)
            if rem:
                off = base + nfull * GC
                pltpu.sync_copy(acc.at[pl.ds(off, rem)],
                                bufA.at[0].at[pl.ds(0, rem)])
                pltpu.sync_copy(bufA.at[0].at[pl.ds(0, rem)],
                                out_h.at[pl.ds(p * R + off, rem), cs])
            plsc.subcore_barrier()

    return k(tabP, tabQ, src3, dst3)


def kernel(x, hidden_attr, W_hid, W_src, W_e1, W_e2, Wp1, Wp2, W_d1, W_d2,
           W_out, enc_src, enc_dst, proc_src, proc_dst, dec_src, dec_dst):
    n_hid = hidden_attr.shape[0]
    n_data = x.shape[0]
    n_layers = Wp1.shape[0]

    xe = _mm(x, W_src, relu=True)
    h = _mm(hidden_attr, W_hid, relu=True)

    P = _mm(xe, W_e1[:C])
    Q = _mm(h, W_e1[C:])
    S = _sc_edge(P, Q, enc_src, enc_dst, n_hid, 1)
    h = _mm(S, W_e2, res=h, relu=True, m_out=n_hid)

    for k in range(n_layers):
        A = _mm(h, Wp1[k][:C])
        B = _mm(h, Wp1[k][C:])
        S = _sc_edge(A, B, proc_src, proc_dst, n_hid, 1)
        h = _mm(S, Wp2[k], res=h, m_out=n_hid)

    Pd = _mm(h, W_d1[:C])
    Qd = _mm(xe, W_d1[C:])
    Sd = _sc_edge(Pd, Qd, dec_src, dec_dst, n_data, 4)
    xd = _mm(Sd, W_d2, res=xe, m_out=n_data)

    return _mm(xd, W_out)
